# baseline (speedup 1.0000x reference)
import functools

import jax
import jax.numpy as jnp
from jax.experimental import pallas as pl
from jax.experimental.pallas import tpu as pltpu

BAND = 10


def _feat_kernel(x_ref, w_ref, wb_ref, xw_ref, xwb_ref):
    xb = x_ref[0]
    xw_ref[0] = jnp.dot(xb, w_ref[...], preferred_element_type=jnp.float32)
    xwb_ref[0] = jnp.dot(xb, wb_ref[...], preferred_element_type=jnp.float32)


def _layer1_kernel(adj_ref, xw_ref, xwb_ref, b_ref, bb_ref, w3_ref, wb3_ref,
                   hw3_ref, hwb3_ref, acc_nl, acc_loc, *, ti, tj, nj):
    i = pl.program_id(1)
    j = pl.program_id(2)

    @pl.when(j == 0)
    def _():
        acc_nl[...] = jnp.zeros_like(acc_nl)
        acc_loc[...] = jnp.zeros_like(acc_loc)

    a = adj_ref[0]
    xw = xw_ref[0, pl.ds(j * tj, tj), :]
    acc_nl[...] += jnp.dot(a, xw, preferred_element_type=jnp.float32)

    @pl.when(jnp.abs(i - j) <= 1)
    def _():
        ri = jax.lax.broadcasted_iota(jnp.int32, (ti, tj), 0) + i * ti
        rj = jax.lax.broadcasted_iota(jnp.int32, (ti, tj), 1) + j * tj
        am = jnp.where(jnp.abs(ri - rj) <= BAND, a, 0.0)
        xwb = xwb_ref[0, pl.ds(j * tj, tj), :]
        acc_loc[...] += jnp.dot(am, xwb, preferred_element_type=jnp.float32)

    @pl.when(j == nj - 1)
    def _():
        h = (jax.nn.relu(acc_nl[...] + b_ref[...])
             + jax.nn.relu(acc_loc[...] + bb_ref[...]))
        hw3_ref[0] = jnp.dot(h, w3_ref[...], preferred_element_type=jnp.float32)
        hwb3_ref[0] = jnp.dot(h, wb3_ref[...], preferred_element_type=jnp.float32)


def _layer2_kernel(adj_ref, hw_ref, hwb_ref, b_ref, bb_ref, wfc_ref, bfc_ref,
                   out_ref, acc_nl, acc_loc, *, ti, tj, nj, n):
    bidx = pl.program_id(0)
    i = pl.program_id(1)
    j = pl.program_id(2)

    @pl.when(j == 0)
    def _():
        acc_nl[...] = jnp.zeros_like(acc_nl)
        acc_loc[...] = jnp.zeros_like(acc_loc)

    a = adj_ref[0]
    hw = hw_ref[0, pl.ds(j * tj, tj), :]
    acc_nl[...] += jnp.dot(a, hw, preferred_element_type=jnp.float32)

    @pl.when(jnp.abs(i - j) <= 1)
    def _():
        ri = jax.lax.broadcasted_iota(jnp.int32, (ti, tj), 0) + i * ti
        rj = jax.lax.broadcasted_iota(jnp.int32, (ti, tj), 1) + j * tj
        am = jnp.where(jnp.abs(ri - rj) <= BAND, a, 0.0)
        hwb = hwb_ref[0, pl.ds(j * tj, tj), :]
        acc_loc[...] += jnp.dot(am, hwb, preferred_element_type=jnp.float32)

    @pl.when(j == nj - 1)
    def _():
        h = (jax.nn.relu(acc_nl[...] + b_ref[...])
             + jax.nn.relu(acc_loc[...] + bb_ref[...]))
        pooled = jnp.sum(h, axis=0, keepdims=True) * (1.0 / n)
        contrib = jnp.dot(pooled, wfc_ref[...], preferred_element_type=jnp.float32)

        @pl.when(i == 0)
        def _():
            out_ref[pl.ds(bidx, 1), :] = bfc_ref[...] + contrib

        @pl.when(i > 0)
        def _():
            out_ref[pl.ds(bidx, 1), :] += contrib


def kernel(x, adj, W1, b1, Wb1, bb1, W3, b3, Wb3, bb3, Wfc, bfc):
    B, N, NFEAT = x.shape
    NH1 = W1.shape[1]
    NH2 = W3.shape[1]
    NCLASS = Wfc.shape[1]

    TI = min(256, N)
    TJ = TI
    NI = N // TI
    NJ = N // TJ

    b1r = b1.reshape(1, NH1)
    bb1r = bb1.reshape(1, NH1)
    b3r = b3.reshape(1, NH2)
    bb3r = bb3.reshape(1, NH2)
    bfcr = bfc.reshape(1, NCLASS)

    xw1, xwb1 = pl.pallas_call(
        _feat_kernel,
        grid=(B, NI),
        in_specs=[
            pl.BlockSpec((1, TI, NFEAT), lambda b, i: (b, i, 0)),
            pl.BlockSpec((NFEAT, NH1), lambda b, i: (0, 0)),
            pl.BlockSpec((NFEAT, NH1), lambda b, i: (0, 0)),
        ],
        out_specs=[
            pl.BlockSpec((1, TI, NH1), lambda b, i: (b, i, 0)),
            pl.BlockSpec((1, TI, NH1), lambda b, i: (b, i, 0)),
        ],
        out_shape=[
            jax.ShapeDtypeStruct((B, N, NH1), jnp.float32),
            jax.ShapeDtypeStruct((B, N, NH1), jnp.float32),
        ],
    )(x, W1, Wb1)

    hw3, hwb3 = pl.pallas_call(
        functools.partial(_layer1_kernel, ti=TI, tj=TJ, nj=NJ),
        grid=(B, NI, NJ),
        in_specs=[
            pl.BlockSpec((1, TI, TJ), lambda b, i, j: (b, i, j)),
            pl.BlockSpec((1, N, NH1), lambda b, i, j: (b, 0, 0)),
            pl.BlockSpec((1, N, NH1), lambda b, i, j: (b, 0, 0)),
            pl.BlockSpec((1, NH1), lambda b, i, j: (0, 0)),
            pl.BlockSpec((1, NH1), lambda b, i, j: (0, 0)),
            pl.BlockSpec((NH1, NH2), lambda b, i, j: (0, 0)),
            pl.BlockSpec((NH1, NH2), lambda b, i, j: (0, 0)),
        ],
        out_specs=[
            pl.BlockSpec((1, TI, NH2), lambda b, i, j: (b, i, 0)),
            pl.BlockSpec((1, TI, NH2), lambda b, i, j: (b, i, 0)),
        ],
        out_shape=[
            jax.ShapeDtypeStruct((B, N, NH2), jnp.float32),
            jax.ShapeDtypeStruct((B, N, NH2), jnp.float32),
        ],
        scratch_shapes=[
            pltpu.VMEM((TI, NH1), jnp.float32),
            pltpu.VMEM((TI, NH1), jnp.float32),
        ],
    )(adj, xw1, xwb1, b1r, bb1r, W3, Wb3)

    out = pl.pallas_call(
        functools.partial(_layer2_kernel, ti=TI, tj=TJ, nj=NJ, n=N),
        grid=(B, NI, NJ),
        in_specs=[
            pl.BlockSpec((1, TI, TJ), lambda b, i, j: (b, i, j)),
            pl.BlockSpec((1, N, NH2), lambda b, i, j: (b, 0, 0)),
            pl.BlockSpec((1, N, NH2), lambda b, i, j: (b, 0, 0)),
            pl.BlockSpec((1, NH2), lambda b, i, j: (0, 0)),
            pl.BlockSpec((1, NH2), lambda b, i, j: (0, 0)),
            pl.BlockSpec((NH2, NCLASS), lambda b, i, j: (0, 0)),
            pl.BlockSpec((1, NCLASS), lambda b, i, j: (0, 0)),
        ],
        out_specs=pl.BlockSpec((B, NCLASS), lambda b, i, j: (0, 0)),
        out_shape=jax.ShapeDtypeStruct((B, NCLASS), jnp.float32),
        scratch_shapes=[
            pltpu.VMEM((TI, NH2), jnp.float32),
            pltpu.VMEM((TI, NH2), jnp.float32),
        ],
    )(adj, hw3, hwb3, b3r, bb3r, Wfc, bfcr)

    return out


# speedup vs baseline: 2.5610x; 2.5610x over previous
import functools

import jax
import jax.numpy as jnp
from jax.experimental import pallas as pl
from jax.experimental.pallas import tpu as pltpu

BAND = 10
CPAD = 128
CROWS = 16


def _feat_kernel(x_ref, w_ref, wb_ref, xw_ref, xwb_ref):
    xb = x_ref[0]
    xw_ref[0] = jnp.dot(xb, w_ref[...], preferred_element_type=jnp.float32)
    xwb_ref[0] = jnp.dot(xb, wb_ref[...], preferred_element_type=jnp.float32)


def _band_local(adj_ref, xwb_ref, loc_ref, i, ti, ni):
    ad = adj_ref[0, :, pl.ds(i * ti, ti)]
    r = jax.lax.broadcasted_iota(jnp.int32, (ti, ti), 0)
    c = jax.lax.broadcasted_iota(jnp.int32, (ti, ti), 1)
    adm = jnp.where(jnp.abs(r - c) <= BAND, ad, 0.0)
    xwb_d = xwb_ref[0, pl.ds(i * ti, ti), :]
    loc_ref[...] = jnp.dot(adm, xwb_d, preferred_element_type=jnp.float32)

    @pl.when(i > 0)
    def _():
        at = adj_ref[0, 0:CROWS, pl.ds(i * ti - CPAD, CPAD)]
        rr = jax.lax.broadcasted_iota(jnp.int32, (CROWS, CPAD), 0)
        cc = jax.lax.broadcasted_iota(jnp.int32, (CROWS, CPAD), 1)
        atm = jnp.where(jnp.abs(rr + CPAD - cc) <= BAND, at, 0.0)
        xwb_t = xwb_ref[0, pl.ds(i * ti - CPAD, CPAD), :]
        loc_ref[0:CROWS, :] += jnp.dot(
            atm, xwb_t, preferred_element_type=jnp.float32)

    @pl.when(i < ni - 1)
    def _():
        ab = adj_ref[0, ti - CROWS:ti, pl.ds((i + 1) * ti, CPAD)]
        rr = jax.lax.broadcasted_iota(jnp.int32, (CROWS, CPAD), 0)
        cc = jax.lax.broadcasted_iota(jnp.int32, (CROWS, CPAD), 1)
        abm = jnp.where(jnp.abs(rr - CROWS - cc) <= BAND, ab, 0.0)
        xwb_b = xwb_ref[0, pl.ds((i + 1) * ti, CPAD), :]
        loc_ref[ti - CROWS:ti, :] += jnp.dot(
            abm, xwb_b, preferred_element_type=jnp.float32)


def _layer1_kernel(adj_ref, xw_ref, xwb_ref, b_ref, bb_ref, w3_ref, wb3_ref,
                   hw3_ref, hwb3_ref, loc_ref, *, ti, ni):
    i = pl.program_id(1)
    nl = jnp.dot(adj_ref[0], xw_ref[0], preferred_element_type=jnp.float32)
    _band_local(adj_ref, xwb_ref, loc_ref, i, ti, ni)
    h = (jax.nn.relu(nl + b_ref[...])
         + jax.nn.relu(loc_ref[...] + bb_ref[...]))
    hw3_ref[0] = jnp.dot(h, w3_ref[...], preferred_element_type=jnp.float32)
    hwb3_ref[0] = jnp.dot(h, wb3_ref[...], preferred_element_type=jnp.float32)


def _layer2_kernel(adj_ref, hw_ref, hwb_ref, b_ref, bb_ref, wfc_ref, bfc_ref,
                   out_ref, loc_ref, *, ti, ni, n):
    bidx = pl.program_id(0)
    i = pl.program_id(1)
    nl = jnp.dot(adj_ref[0], hw_ref[0], preferred_element_type=jnp.float32)
    _band_local(adj_ref, hwb_ref, loc_ref, i, ti, ni)
    h = (jax.nn.relu(nl + b_ref[...])
         + jax.nn.relu(loc_ref[...] + bb_ref[...]))
    pooled = jnp.sum(h, axis=0, keepdims=True) * (1.0 / n)
    contrib = jnp.dot(pooled, wfc_ref[...], preferred_element_type=jnp.float32)

    @pl.when(i == 0)
    def _():
        out_ref[pl.ds(bidx, 1), :] = bfc_ref[...] + contrib

    @pl.when(i > 0)
    def _():
        out_ref[pl.ds(bidx, 1), :] += contrib


def kernel(x, adj, W1, b1, Wb1, bb1, W3, b3, Wb3, bb3, Wfc, bfc):
    B, N, NFEAT = x.shape
    NH1 = W1.shape[1]
    NH2 = W3.shape[1]
    NCLASS = Wfc.shape[1]

    TI = min(256, N)
    NI = N // TI

    b1r = b1.reshape(1, NH1)
    bb1r = bb1.reshape(1, NH1)
    b3r = b3.reshape(1, NH2)
    bb3r = bb3.reshape(1, NH2)
    bfcr = bfc.reshape(1, NCLASS)

    xw1, xwb1 = pl.pallas_call(
        _feat_kernel,
        grid=(B, NI),
        in_specs=[
            pl.BlockSpec((1, TI, NFEAT), lambda b, i: (b, i, 0)),
            pl.BlockSpec((NFEAT, NH1), lambda b, i: (0, 0)),
            pl.BlockSpec((NFEAT, NH1), lambda b, i: (0, 0)),
        ],
        out_specs=[
            pl.BlockSpec((1, TI, NH1), lambda b, i: (b, i, 0)),
            pl.BlockSpec((1, TI, NH1), lambda b, i: (b, i, 0)),
        ],
        out_shape=[
            jax.ShapeDtypeStruct((B, N, NH1), jnp.float32),
            jax.ShapeDtypeStruct((B, N, NH1), jnp.float32),
        ],
    )(x, W1, Wb1)

    hw3, hwb3 = pl.pallas_call(
        functools.partial(_layer1_kernel, ti=TI, ni=NI),
        grid=(B, NI),
        in_specs=[
            pl.BlockSpec((1, TI, N), lambda b, i: (b, i, 0)),
            pl.BlockSpec((1, N, NH1), lambda b, i: (b, 0, 0)),
            pl.BlockSpec((1, N, NH1), lambda b, i: (b, 0, 0)),
            pl.BlockSpec((1, NH1), lambda b, i: (0, 0)),
            pl.BlockSpec((1, NH1), lambda b, i: (0, 0)),
            pl.BlockSpec((NH1, NH2), lambda b, i: (0, 0)),
            pl.BlockSpec((NH1, NH2), lambda b, i: (0, 0)),
        ],
        out_specs=[
            pl.BlockSpec((1, TI, NH2), lambda b, i: (b, i, 0)),
            pl.BlockSpec((1, TI, NH2), lambda b, i: (b, i, 0)),
        ],
        out_shape=[
            jax.ShapeDtypeStruct((B, N, NH2), jnp.float32),
            jax.ShapeDtypeStruct((B, N, NH2), jnp.float32),
        ],
        scratch_shapes=[
            pltpu.VMEM((TI, NH1), jnp.float32),
        ],
    )(adj, xw1, xwb1, b1r, bb1r, W3, Wb3)

    out = pl.pallas_call(
        functools.partial(_layer2_kernel, ti=TI, ni=NI, n=N),
        grid=(B, NI),
        in_specs=[
            pl.BlockSpec((1, TI, N), lambda b, i: (b, i, 0)),
            pl.BlockSpec((1, N, NH2), lambda b, i: (b, 0, 0)),
            pl.BlockSpec((1, N, NH2), lambda b, i: (b, 0, 0)),
            pl.BlockSpec((1, NH2), lambda b, i: (0, 0)),
            pl.BlockSpec((1, NH2), lambda b, i: (0, 0)),
            pl.BlockSpec((NH2, NCLASS), lambda b, i: (0, 0)),
            pl.BlockSpec((1, NCLASS), lambda b, i: (0, 0)),
        ],
        out_specs=pl.BlockSpec((B, NCLASS), lambda b, i: (0, 0)),
        out_shape=jax.ShapeDtypeStruct((B, NCLASS), jnp.float32),
        scratch_shapes=[
            pltpu.VMEM((TI, NH2), jnp.float32),
        ],
    )(adj, hw3, hwb3, b3r, bb3r, Wfc, bfcr)

    return out


# speedup vs baseline: 3.2883x; 1.2840x over previous
import functools

import jax
import jax.numpy as jnp
from jax.experimental import pallas as pl
from jax.experimental.pallas import tpu as pltpu

BAND = 10
CPAD = 128
CROWS = 16


def _feat_kernel(x_ref, w_ref, wb_ref, xw_ref, xwb_ref):
    xb = x_ref[0]
    xw_ref[0] = jnp.dot(xb, w_ref[...], preferred_element_type=jnp.float32)
    xwb_ref[0] = jnp.dot(xb, wb_ref[...], preferred_element_type=jnp.float32)


def _band_local(adj_ref, xwb_ref, loc_ref, i, ti, ni):
    ad = adj_ref[0, :, pl.ds(i * ti, ti)]
    r = jax.lax.broadcasted_iota(jnp.int32, (ti, ti), 0)
    c = jax.lax.broadcasted_iota(jnp.int32, (ti, ti), 1)
    adm = jnp.where(jnp.abs(r - c) <= BAND, ad, 0.0)
    xwb_d = xwb_ref[0, pl.ds(i * ti, ti), :]
    loc_ref[...] = jnp.dot(adm, xwb_d, preferred_element_type=jnp.float32)

    @pl.when(i > 0)
    def _():
        at = adj_ref[0, 0:CROWS, pl.ds(i * ti - CPAD, CPAD)]
        rr = jax.lax.broadcasted_iota(jnp.int32, (CROWS, CPAD), 0)
        cc = jax.lax.broadcasted_iota(jnp.int32, (CROWS, CPAD), 1)
        atm = jnp.where(jnp.abs(rr + CPAD - cc) <= BAND, at, 0.0)
        xwb_t = xwb_ref[0, pl.ds(i * ti - CPAD, CPAD), :]
        loc_ref[0:CROWS, :] += jnp.dot(
            atm, xwb_t, preferred_element_type=jnp.float32)

    @pl.when(i < ni - 1)
    def _():
        ab = adj_ref[0, ti - CROWS:ti, pl.ds((i + 1) * ti, CPAD)]
        rr = jax.lax.broadcasted_iota(jnp.int32, (CROWS, CPAD), 0)
        cc = jax.lax.broadcasted_iota(jnp.int32, (CROWS, CPAD), 1)
        abm = jnp.where(jnp.abs(rr - CROWS - cc) <= BAND, ab, 0.0)
        xwb_b = xwb_ref[0, pl.ds((i + 1) * ti, CPAD), :]
        loc_ref[ti - CROWS:ti, :] += jnp.dot(
            abm, xwb_b, preferred_element_type=jnp.float32)


def _layer1_kernel(adj_ref, xw_ref, xwb_ref, b_ref, bb_ref, w3_ref, wb3_ref,
                   hw3_ref, hwb3_ref, loc_ref, *, ti, ni):
    i = pl.program_id(1)
    nl = jnp.dot(adj_ref[0], xw_ref[0], preferred_element_type=jnp.float32)
    _band_local(adj_ref, xwb_ref, loc_ref, i, ti, ni)
    h = (jax.nn.relu(nl + b_ref[...])
         + jax.nn.relu(loc_ref[...] + bb_ref[...]))
    hw3_ref[0] = jnp.dot(h, w3_ref[...], preferred_element_type=jnp.float32)
    hwb3_ref[0] = jnp.dot(h, wb3_ref[...], preferred_element_type=jnp.float32)


def _layer2_kernel(adj_ref, hw_ref, hwb_ref, b_ref, bb_ref, wfc_ref, bfc_ref,
                   out_ref, loc_ref, *, ti, ni, n):
    bidx = pl.program_id(0)
    i = pl.program_id(1)
    nl = jnp.dot(adj_ref[0], hw_ref[0], preferred_element_type=jnp.float32)
    _band_local(adj_ref, hwb_ref, loc_ref, i, ti, ni)
    h = (jax.nn.relu(nl + b_ref[...])
         + jax.nn.relu(loc_ref[...] + bb_ref[...]))
    pooled = jnp.sum(h, axis=0, keepdims=True) * (1.0 / n)
    contrib = jnp.dot(pooled, wfc_ref[...], preferred_element_type=jnp.float32)

    @pl.when(i == 0)
    def _():
        out_ref[pl.ds(bidx, 1), :] = bfc_ref[...] + contrib

    @pl.when(i > 0)
    def _():
        out_ref[pl.ds(bidx, 1), :] += contrib


def kernel(x, adj, W1, b1, Wb1, bb1, W3, b3, Wb3, bb3, Wfc, bfc):
    B, N, NFEAT = x.shape
    NH1 = W1.shape[1]
    NH2 = W3.shape[1]
    NCLASS = Wfc.shape[1]

    TI = min(512, N)
    NI = N // TI

    b1r = b1.reshape(1, NH1)
    bb1r = bb1.reshape(1, NH1)
    b3r = b3.reshape(1, NH2)
    bb3r = bb3.reshape(1, NH2)
    bfcr = bfc.reshape(1, NCLASS)

    xw1, xwb1 = pl.pallas_call(
        _feat_kernel,
        grid=(B, NI),
        in_specs=[
            pl.BlockSpec((1, TI, NFEAT), lambda b, i: (b, i, 0)),
            pl.BlockSpec((NFEAT, NH1), lambda b, i: (0, 0)),
            pl.BlockSpec((NFEAT, NH1), lambda b, i: (0, 0)),
        ],
        out_specs=[
            pl.BlockSpec((1, TI, NH1), lambda b, i: (b, i, 0)),
            pl.BlockSpec((1, TI, NH1), lambda b, i: (b, i, 0)),
        ],
        out_shape=[
            jax.ShapeDtypeStruct((B, N, NH1), jnp.float32),
            jax.ShapeDtypeStruct((B, N, NH1), jnp.float32),
        ],
    )(x, W1, Wb1)

    hw3, hwb3 = pl.pallas_call(
        functools.partial(_layer1_kernel, ti=TI, ni=NI),
        grid=(B, NI),
        in_specs=[
            pl.BlockSpec((1, TI, N), lambda b, i: (b, i, 0)),
            pl.BlockSpec((1, N, NH1), lambda b, i: (b, 0, 0)),
            pl.BlockSpec((1, N, NH1), lambda b, i: (b, 0, 0)),
            pl.BlockSpec((1, NH1), lambda b, i: (0, 0)),
            pl.BlockSpec((1, NH1), lambda b, i: (0, 0)),
            pl.BlockSpec((NH1, NH2), lambda b, i: (0, 0)),
            pl.BlockSpec((NH1, NH2), lambda b, i: (0, 0)),
        ],
        out_specs=[
            pl.BlockSpec((1, TI, NH2), lambda b, i: (b, i, 0)),
            pl.BlockSpec((1, TI, NH2), lambda b, i: (b, i, 0)),
        ],
        out_shape=[
            jax.ShapeDtypeStruct((B, N, NH2), jnp.float32),
            jax.ShapeDtypeStruct((B, N, NH2), jnp.float32),
        ],
        scratch_shapes=[
            pltpu.VMEM((TI, NH1), jnp.float32),
        ],
    )(adj, xw1, xwb1, b1r, bb1r, W3, Wb3)

    out = pl.pallas_call(
        functools.partial(_layer2_kernel, ti=TI, ni=NI, n=N),
        grid=(B, NI),
        in_specs=[
            pl.BlockSpec((1, TI, N), lambda b, i: (b, i, 0)),
            pl.BlockSpec((1, N, NH2), lambda b, i: (b, 0, 0)),
            pl.BlockSpec((1, N, NH2), lambda b, i: (b, 0, 0)),
            pl.BlockSpec((1, NH2), lambda b, i: (0, 0)),
            pl.BlockSpec((1, NH2), lambda b, i: (0, 0)),
            pl.BlockSpec((NH2, NCLASS), lambda b, i: (0, 0)),
            pl.BlockSpec((1, NCLASS), lambda b, i: (0, 0)),
        ],
        out_specs=pl.BlockSpec((B, NCLASS), lambda b, i: (0, 0)),
        out_shape=jax.ShapeDtypeStruct((B, NCLASS), jnp.float32),
        scratch_shapes=[
            pltpu.VMEM((TI, NH2), jnp.float32),
        ],
    )(adj, hw3, hwb3, b3r, bb3r, Wfc, bfcr)

    return out


# speedup vs baseline: 3.5279x; 1.0729x over previous
import functools

import jax
import jax.numpy as jnp
from jax.experimental import pallas as pl
from jax.experimental.pallas import tpu as pltpu

BAND = 10
CPAD = 128
CROWS = 16


def _feat_kernel(x_ref, w_ref, wb_ref, xw_ref, xwb_ref):
    xb = x_ref[0]
    xw = jnp.dot(xb, w_ref[...], preferred_element_type=jnp.float32)
    xw_ref[0] = xw.astype(jnp.bfloat16)
    xwb_ref[0] = jnp.dot(xb, wb_ref[...], preferred_element_type=jnp.float32)


def _band_local(adj_ref, xwb_ref, loc_ref, i, ti, ni):
    ad = adj_ref[0, :, pl.ds(i * ti, ti)]
    r = jax.lax.broadcasted_iota(jnp.int32, (ti, ti), 0)
    c = jax.lax.broadcasted_iota(jnp.int32, (ti, ti), 1)
    adm = jnp.where(jnp.abs(r - c) <= BAND, ad, 0.0)
    xwb_d = xwb_ref[0, pl.ds(i * ti, ti), :]
    loc_ref[...] = jnp.dot(adm, xwb_d, preferred_element_type=jnp.float32)

    @pl.when(i > 0)
    def _():
        at = adj_ref[0, 0:CROWS, pl.ds(i * ti - CPAD, CPAD)]
        rr = jax.lax.broadcasted_iota(jnp.int32, (CROWS, CPAD), 0)
        cc = jax.lax.broadcasted_iota(jnp.int32, (CROWS, CPAD), 1)
        atm = jnp.where(jnp.abs(rr + CPAD - cc) <= BAND, at, 0.0)
        xwb_t = xwb_ref[0, pl.ds(i * ti - CPAD, CPAD), :]
        loc_ref[0:CROWS, :] += jnp.dot(
            atm, xwb_t, preferred_element_type=jnp.float32)

    @pl.when(i < ni - 1)
    def _():
        ab = adj_ref[0, ti - CROWS:ti, pl.ds((i + 1) * ti, CPAD)]
        rr = jax.lax.broadcasted_iota(jnp.int32, (CROWS, CPAD), 0)
        cc = jax.lax.broadcasted_iota(jnp.int32, (CROWS, CPAD), 1)
        abm = jnp.where(jnp.abs(rr - CROWS - cc) <= BAND, ab, 0.0)
        xwb_b = xwb_ref[0, pl.ds((i + 1) * ti, CPAD), :]
        loc_ref[ti - CROWS:ti, :] += jnp.dot(
            abm, xwb_b, preferred_element_type=jnp.float32)


def _layer1_kernel(adj_ref, xw_ref, xwb_ref, b_ref, bb_ref, w3_ref, wb3_ref,
                   hw3_ref, hwb3_ref, loc_ref, *, ti, ni):
    i = pl.program_id(1)
    nl = jnp.dot(adj_ref[0].astype(jnp.bfloat16), xw_ref[0],
                 preferred_element_type=jnp.float32)
    _band_local(adj_ref, xwb_ref, loc_ref, i, ti, ni)
    h = (jax.nn.relu(nl + b_ref[...])
         + jax.nn.relu(loc_ref[...] + bb_ref[...]))
    hw3 = jnp.dot(h, w3_ref[...], preferred_element_type=jnp.float32)
    hw3_ref[0] = hw3.astype(jnp.bfloat16)
    hwb3_ref[0] = jnp.dot(h, wb3_ref[...], preferred_element_type=jnp.float32)


def _layer2_kernel(adj_ref, hw_ref, hwb_ref, b_ref, bb_ref, wfc_ref, bfc_ref,
                   out_ref, loc_ref, *, ti, ni, n):
    bidx = pl.program_id(0)
    i = pl.program_id(1)
    nl = jnp.dot(adj_ref[0].astype(jnp.bfloat16), hw_ref[0],
                 preferred_element_type=jnp.float32)
    _band_local(adj_ref, hwb_ref, loc_ref, i, ti, ni)
    h = (jax.nn.relu(nl + b_ref[...])
         + jax.nn.relu(loc_ref[...] + bb_ref[...]))
    pooled = jnp.sum(h, axis=0, keepdims=True) * (1.0 / n)
    contrib = jnp.dot(pooled, wfc_ref[...], preferred_element_type=jnp.float32)

    @pl.when(i == 0)
    def _():
        out_ref[pl.ds(bidx, 1), :] = bfc_ref[...] + contrib

    @pl.when(i > 0)
    def _():
        out_ref[pl.ds(bidx, 1), :] += contrib


def kernel(x, adj, W1, b1, Wb1, bb1, W3, b3, Wb3, bb3, Wfc, bfc):
    B, N, NFEAT = x.shape
    NH1 = W1.shape[1]
    NH2 = W3.shape[1]
    NCLASS = Wfc.shape[1]

    TI = min(512, N)
    NI = N // TI

    b1r = b1.reshape(1, NH1)
    bb1r = bb1.reshape(1, NH1)
    b3r = b3.reshape(1, NH2)
    bb3r = bb3.reshape(1, NH2)
    bfcr = bfc.reshape(1, NCLASS)

    xw1, xwb1 = pl.pallas_call(
        _feat_kernel,
        grid=(B, NI),
        in_specs=[
            pl.BlockSpec((1, TI, NFEAT), lambda b, i: (b, i, 0)),
            pl.BlockSpec((NFEAT, NH1), lambda b, i: (0, 0)),
            pl.BlockSpec((NFEAT, NH1), lambda b, i: (0, 0)),
        ],
        out_specs=[
            pl.BlockSpec((1, TI, NH1), lambda b, i: (b, i, 0)),
            pl.BlockSpec((1, TI, NH1), lambda b, i: (b, i, 0)),
        ],
        out_shape=[
            jax.ShapeDtypeStruct((B, N, NH1), jnp.bfloat16),
            jax.ShapeDtypeStruct((B, N, NH1), jnp.float32),
        ],
    )(x, W1, Wb1)

    hw3, hwb3 = pl.pallas_call(
        functools.partial(_layer1_kernel, ti=TI, ni=NI),
        grid=(B, NI),
        in_specs=[
            pl.BlockSpec((1, TI, N), lambda b, i: (b, i, 0)),
            pl.BlockSpec((1, N, NH1), lambda b, i: (b, 0, 0)),
            pl.BlockSpec((1, N, NH1), lambda b, i: (b, 0, 0)),
            pl.BlockSpec((1, NH1), lambda b, i: (0, 0)),
            pl.BlockSpec((1, NH1), lambda b, i: (0, 0)),
            pl.BlockSpec((NH1, NH2), lambda b, i: (0, 0)),
            pl.BlockSpec((NH1, NH2), lambda b, i: (0, 0)),
        ],
        out_specs=[
            pl.BlockSpec((1, TI, NH2), lambda b, i: (b, i, 0)),
            pl.BlockSpec((1, TI, NH2), lambda b, i: (b, i, 0)),
        ],
        out_shape=[
            jax.ShapeDtypeStruct((B, N, NH2), jnp.bfloat16),
            jax.ShapeDtypeStruct((B, N, NH2), jnp.float32),
        ],
        scratch_shapes=[
            pltpu.VMEM((TI, NH1), jnp.float32),
        ],
    )(adj, xw1, xwb1, b1r, bb1r, W3, Wb3)

    out = pl.pallas_call(
        functools.partial(_layer2_kernel, ti=TI, ni=NI, n=N),
        grid=(B, NI),
        in_specs=[
            pl.BlockSpec((1, TI, N), lambda b, i: (b, i, 0)),
            pl.BlockSpec((1, N, NH2), lambda b, i: (b, 0, 0)),
            pl.BlockSpec((1, N, NH2), lambda b, i: (b, 0, 0)),
            pl.BlockSpec((1, NH2), lambda b, i: (0, 0)),
            pl.BlockSpec((1, NH2), lambda b, i: (0, 0)),
            pl.BlockSpec((NH2, NCLASS), lambda b, i: (0, 0)),
            pl.BlockSpec((1, NCLASS), lambda b, i: (0, 0)),
        ],
        out_specs=pl.BlockSpec((B, NCLASS), lambda b, i: (0, 0)),
        out_shape=jax.ShapeDtypeStruct((B, NCLASS), jnp.float32),
        scratch_shapes=[
            pltpu.VMEM((TI, NH2), jnp.float32),
        ],
    )(adj, hw3, hwb3, b3r, bb3r, Wfc, bfcr)

    return out


# speedup vs baseline: 4.3176x; 1.2238x over previous
import functools

import jax
import jax.numpy as jnp
from jax.experimental import pallas as pl
from jax.experimental.pallas import tpu as pltpu

BAND = 10
CPAD = 128
CROWS = 16


def _band_local(adj_ref, src, loc_ref, i, ti, ni, nh):
    ad = adj_ref[0, :, pl.ds(i * ti, ti)]
    r = jax.lax.broadcasted_iota(jnp.int32, (ti, ti), 0)
    c = jax.lax.broadcasted_iota(jnp.int32, (ti, ti), 1)
    adm = jnp.where(jnp.abs(r - c) <= BAND, ad, 0.0)
    src_d = src[pl.ds(i * ti, ti), 0:nh]
    loc_ref[:, 0:nh] = jnp.dot(adm, src_d, preferred_element_type=jnp.float32)

    @pl.when(i > 0)
    def _():
        at = adj_ref[0, 0:CROWS, pl.ds(i * ti - CPAD, CPAD)]
        rr = jax.lax.broadcasted_iota(jnp.int32, (CROWS, CPAD), 0)
        cc = jax.lax.broadcasted_iota(jnp.int32, (CROWS, CPAD), 1)
        atm = jnp.where(jnp.abs(rr + CPAD - cc) <= BAND, at, 0.0)
        src_t = src[pl.ds(i * ti - CPAD, CPAD), 0:nh]
        loc_ref[0:CROWS, 0:nh] += jnp.dot(
            atm, src_t, preferred_element_type=jnp.float32)

    @pl.when(i < ni - 1)
    def _():
        ab = adj_ref[0, ti - CROWS:ti, pl.ds((i + 1) * ti, CPAD)]
        rr = jax.lax.broadcasted_iota(jnp.int32, (CROWS, CPAD), 0)
        cc = jax.lax.broadcasted_iota(jnp.int32, (CROWS, CPAD), 1)
        abm = jnp.where(jnp.abs(rr - CROWS - cc) <= BAND, ab, 0.0)
        src_b = src[pl.ds((i + 1) * ti, CPAD), 0:nh]
        loc_ref[ti - CROWS:ti, 0:nh] += jnp.dot(
            abm, src_b, preferred_element_type=jnp.float32)


def _gcn_kernel(x_ref, adj_ref, w1_ref, b1_ref, wb1_ref, bb1_ref,
                w3_ref, b3_ref, wb3_ref, bb3_ref, wfc_ref, bfc_ref,
                out_ref, xw, xwb, hw, hwb, loc_ref,
                *, ti, ni, n, nh1, nh2):
    bidx = pl.program_id(0)
    s = pl.program_id(1)
    i = pl.program_id(2)
    rows = pl.ds(i * ti, ti)

    @pl.when(s == 0)
    def _feat():
        xb = x_ref[0]
        t = jnp.dot(xb, w1_ref[...], preferred_element_type=jnp.float32)
        xw[rows, :] = t.astype(jnp.bfloat16)
        xwb[rows, :] = jnp.dot(xb, wb1_ref[...],
                               preferred_element_type=jnp.float32)

    @pl.when(s == 1)
    def _layer1():
        nl = jnp.dot(adj_ref[0].astype(jnp.bfloat16), xw[...],
                     preferred_element_type=jnp.float32)
        _band_local(adj_ref, xwb, loc_ref, i, ti, ni, nh1)
        h = (jax.nn.relu(nl + b1_ref[...])
             + jax.nn.relu(loc_ref[...] + bb1_ref[...]))
        t = jnp.dot(h, w3_ref[...], preferred_element_type=jnp.float32)
        hw[rows, :] = t.astype(jnp.bfloat16)
        hwb[rows, :] = jnp.dot(h, wb3_ref[...],
                               preferred_element_type=jnp.float32)

    @pl.when(s == 2)
    def _layer2():
        nl = jnp.dot(adj_ref[0].astype(jnp.bfloat16), hw[...],
                     preferred_element_type=jnp.float32)
        _band_local(adj_ref, hwb, loc_ref, i, ti, ni, nh2)
        h = (jax.nn.relu(nl + b3_ref[...])
             + jax.nn.relu(loc_ref[:, 0:nh2] + bb3_ref[...]))
        pooled = jnp.sum(h, axis=0, keepdims=True) * (1.0 / n)
        contrib = jnp.dot(pooled, wfc_ref[...],
                          preferred_element_type=jnp.float32)

        @pl.when(i == 0)
        def _():
            out_ref[pl.ds(bidx, 1), :] = bfc_ref[...] + contrib

        @pl.when(i > 0)
        def _():
            out_ref[pl.ds(bidx, 1), :] += contrib


def kernel(x, adj, W1, b1, Wb1, bb1, W3, b3, Wb3, bb3, Wfc, bfc):
    B, N, NFEAT = x.shape
    NH1 = W1.shape[1]
    NH2 = W3.shape[1]
    NCLASS = Wfc.shape[1]

    TI = min(512, N)
    NI = N // TI

    b1r = b1.reshape(1, NH1)
    bb1r = bb1.reshape(1, NH1)
    b3r = b3.reshape(1, NH2)
    bb3r = bb3.reshape(1, NH2)
    bfcr = bfc.reshape(1, NCLASS)

    out = pl.pallas_call(
        functools.partial(_gcn_kernel, ti=TI, ni=NI, n=N, nh1=NH1, nh2=NH2),
        grid=(B, 3, NI),
        in_specs=[
            pl.BlockSpec((1, TI, NFEAT),
                         lambda b, s, i: (b, jnp.where(s == 0, i, 0), 0)),
            pl.BlockSpec((1, TI, N),
                         lambda b, s, i: (b, jnp.where(s == 0, 0, i), 0)),
            pl.BlockSpec((NFEAT, NH1), lambda b, s, i: (0, 0)),
            pl.BlockSpec((1, NH1), lambda b, s, i: (0, 0)),
            pl.BlockSpec((NFEAT, NH1), lambda b, s, i: (0, 0)),
            pl.BlockSpec((1, NH1), lambda b, s, i: (0, 0)),
            pl.BlockSpec((NH1, NH2), lambda b, s, i: (0, 0)),
            pl.BlockSpec((1, NH2), lambda b, s, i: (0, 0)),
            pl.BlockSpec((NH1, NH2), lambda b, s, i: (0, 0)),
            pl.BlockSpec((1, NH2), lambda b, s, i: (0, 0)),
            pl.BlockSpec((NH2, NCLASS), lambda b, s, i: (0, 0)),
            pl.BlockSpec((1, NCLASS), lambda b, s, i: (0, 0)),
        ],
        out_specs=pl.BlockSpec((B, NCLASS), lambda b, s, i: (0, 0)),
        out_shape=jax.ShapeDtypeStruct((B, NCLASS), jnp.float32),
        scratch_shapes=[
            pltpu.VMEM((N, NH1), jnp.bfloat16),
            pltpu.VMEM((N, NH1), jnp.float32),
            pltpu.VMEM((N, NH2), jnp.bfloat16),
            pltpu.VMEM((N, NH2), jnp.float32),
            pltpu.VMEM((TI, NH1), jnp.float32),
        ],
    )(x, adj, W1, b1r, Wb1, bb1r, W3, b3r, Wb3, bb3r, Wfc, bfcr)

    return out


# speedup vs baseline: 4.3778x; 1.0140x over previous
import functools

import jax
import jax.numpy as jnp
from jax.experimental import pallas as pl
from jax.experimental.pallas import tpu as pltpu

BAND = 10
CPAD = 128
CROWS = 16


def _band_local(get_tile, src, loc_ref, i, ti, ni, nh):
    ad = get_tile(0, ti, i * ti, ti)
    r = jax.lax.broadcasted_iota(jnp.int32, (ti, ti), 0)
    c = jax.lax.broadcasted_iota(jnp.int32, (ti, ti), 1)
    adm = jnp.where(jnp.abs(r - c) <= BAND, ad, jnp.bfloat16(0))
    src_d = src[pl.ds(i * ti, ti), 0:nh]
    loc_ref[:, 0:nh] = jnp.dot(adm, src_d, preferred_element_type=jnp.float32)

    @pl.when(i > 0)
    def _():
        at = get_tile(0, CROWS, i * ti - CPAD, CPAD)
        rr = jax.lax.broadcasted_iota(jnp.int32, (CROWS, CPAD), 0)
        cc = jax.lax.broadcasted_iota(jnp.int32, (CROWS, CPAD), 1)
        atm = jnp.where(jnp.abs(rr + CPAD - cc) <= BAND, at, jnp.bfloat16(0))
        src_t = src[pl.ds(i * ti - CPAD, CPAD), 0:nh]
        loc_ref[0:CROWS, 0:nh] += jnp.dot(
            atm, src_t, preferred_element_type=jnp.float32)

    @pl.when(i < ni - 1)
    def _():
        ab = get_tile(ti - CROWS, CROWS, (i + 1) * ti, CPAD)
        rr = jax.lax.broadcasted_iota(jnp.int32, (CROWS, CPAD), 0)
        cc = jax.lax.broadcasted_iota(jnp.int32, (CROWS, CPAD), 1)
        abm = jnp.where(jnp.abs(rr - CROWS - cc) <= BAND, ab, jnp.bfloat16(0))
        src_b = src[pl.ds((i + 1) * ti, CPAD), 0:nh]
        loc_ref[ti - CROWS:ti, 0:nh] += jnp.dot(
            abm, src_b, preferred_element_type=jnp.float32)


def _gcn_kernel(x_ref, adj_ref, w1_ref, b1_ref, wb1_ref, bb1_ref,
                w3_ref, b3_ref, wb3_ref, bb3_ref, wfc_ref, bfc_ref,
                out_ref, abf, xw, xwb, hw, hwb, loc_ref,
                *, ti, ni, n, nh1, nh2):
    bidx = pl.program_id(0)
    s = pl.program_id(1)
    i = pl.program_id(2)
    rows = pl.ds(i * ti, ti)

    @pl.when(s == 0)
    def _feat():
        xb = x_ref[0].astype(jnp.bfloat16)
        t = jnp.dot(xb, w1_ref[...].astype(jnp.bfloat16),
                    preferred_element_type=jnp.float32)
        xw[rows, :] = t.astype(jnp.bfloat16)
        t2 = jnp.dot(xb, wb1_ref[...].astype(jnp.bfloat16),
                     preferred_element_type=jnp.float32)
        xwb[rows, :] = t2.astype(jnp.bfloat16)

    @pl.when(s == 1)
    def _layer1():
        a16 = adj_ref[0].astype(jnp.bfloat16)
        abf[rows, :] = a16
        nl = jnp.dot(a16, xw[...], preferred_element_type=jnp.float32)

        def tile(r0, rlen, c0, clen):
            return adj_ref[0, pl.ds(r0, rlen),
                           pl.ds(c0, clen)].astype(jnp.bfloat16)

        _band_local(tile, xwb, loc_ref, i, ti, ni, nh1)
        h = (jax.nn.relu(nl + b1_ref[...])
             + jax.nn.relu(loc_ref[...] + bb1_ref[...]))
        h16 = h.astype(jnp.bfloat16)
        t = jnp.dot(h16, w3_ref[...].astype(jnp.bfloat16),
                    preferred_element_type=jnp.float32)
        hw[rows, :] = t.astype(jnp.bfloat16)
        t2 = jnp.dot(h16, wb3_ref[...].astype(jnp.bfloat16),
                     preferred_element_type=jnp.float32)
        hwb[rows, :] = t2.astype(jnp.bfloat16)

    @pl.when(s == 2)
    def _layer2():
        nl = jnp.dot(abf[rows, :], hw[...], preferred_element_type=jnp.float32)

        def tile(r0, rlen, c0, clen):
            return abf[pl.ds(i * ti + r0, rlen), pl.ds(c0, clen)]

        _band_local(tile, hwb, loc_ref, i, ti, ni, nh2)
        h = (jax.nn.relu(nl + b3_ref[...])
             + jax.nn.relu(loc_ref[:, 0:nh2] + bb3_ref[...]))
        pooled = jnp.sum(h, axis=0, keepdims=True) * (1.0 / n)
        contrib = jnp.dot(pooled, wfc_ref[...],
                          preferred_element_type=jnp.float32)

        @pl.when(i == 0)
        def _():
            out_ref[pl.ds(bidx, 1), :] = bfc_ref[...] + contrib

        @pl.when(i > 0)
        def _():
            out_ref[pl.ds(bidx, 1), :] += contrib


def kernel(x, adj, W1, b1, Wb1, bb1, W3, b3, Wb3, bb3, Wfc, bfc):
    B, N, NFEAT = x.shape
    NH1 = W1.shape[1]
    NH2 = W3.shape[1]
    NCLASS = Wfc.shape[1]

    TI = min(512, N)
    NI = N // TI

    b1r = b1.reshape(1, NH1)
    bb1r = bb1.reshape(1, NH1)
    b3r = b3.reshape(1, NH2)
    bb3r = bb3.reshape(1, NH2)
    bfcr = bfc.reshape(1, NCLASS)

    out = pl.pallas_call(
        functools.partial(_gcn_kernel, ti=TI, ni=NI, n=N, nh1=NH1, nh2=NH2),
        grid=(B, 3, NI),
        in_specs=[
            pl.BlockSpec((1, TI, NFEAT),
                         lambda b, s, i: (b, jnp.where(s == 0, i, 0), 0)),
            pl.BlockSpec((1, TI, N),
                         lambda b, s, i, _ni=NI: (
                             b,
                             jnp.where(s == 0, 0,
                                       jnp.where(s == 1, i, _ni - 1)),
                             0)),
            pl.BlockSpec((NFEAT, NH1), lambda b, s, i: (0, 0)),
            pl.BlockSpec((1, NH1), lambda b, s, i: (0, 0)),
            pl.BlockSpec((NFEAT, NH1), lambda b, s, i: (0, 0)),
            pl.BlockSpec((1, NH1), lambda b, s, i: (0, 0)),
            pl.BlockSpec((NH1, NH2), lambda b, s, i: (0, 0)),
            pl.BlockSpec((1, NH2), lambda b, s, i: (0, 0)),
            pl.BlockSpec((NH1, NH2), lambda b, s, i: (0, 0)),
            pl.BlockSpec((1, NH2), lambda b, s, i: (0, 0)),
            pl.BlockSpec((NH2, NCLASS), lambda b, s, i: (0, 0)),
            pl.BlockSpec((1, NCLASS), lambda b, s, i: (0, 0)),
        ],
        out_specs=pl.BlockSpec((B, NCLASS), lambda b, s, i: (0, 0)),
        out_shape=jax.ShapeDtypeStruct((B, NCLASS), jnp.float32),
        scratch_shapes=[
            pltpu.VMEM((N, N), jnp.bfloat16),
            pltpu.VMEM((N, NH1), jnp.bfloat16),
            pltpu.VMEM((N, NH1), jnp.bfloat16),
            pltpu.VMEM((N, NH2), jnp.bfloat16),
            pltpu.VMEM((N, NH2), jnp.bfloat16),
            pltpu.VMEM((TI, NH1), jnp.float32),
        ],
    )(x, adj, W1, b1r, Wb1, bb1r, W3, b3r, Wb3, bb3r, Wfc, bfcr)

    return out
